# baseline (speedup 1.0000x reference)
import functools

import jax
import jax.numpy as jnp
from jax import lax
from jax.experimental import pallas as pl
from jax.experimental.pallas import tpu as pltpu
from jax.experimental.pallas import tpu_sc as plsc


def _argmin_body(x_ref, c_ref, assign_ref, counts_out_ref, xg_ref, c2_scr, *,
                 num_blocks, num_clusters, bn):
    i = pl.program_id(0)

    @pl.when(i == 0)
    def _init():
        cc = c_ref[...]
        c2_scr[...] = jnp.broadcast_to(
            jnp.sum(cc * cc, axis=1, keepdims=True), c2_scr.shape)
        counts_out_ref[...] = jnp.zeros_like(counts_out_ref)

    x = x_ref[...]
    scores = lax.dot_general(
        c_ref[...], x, (((1,), (1,)), ((), ())),
        preferred_element_type=jnp.float32)
    val = scores - 0.5 * c2_scr[:, 0:1]
    mx = jnp.max(val, axis=0, keepdims=True)
    iota_c = lax.broadcasted_iota(jnp.int32, (num_clusters, bn), 0)
    assign = jnp.min(jnp.where(val == mx, iota_c, num_clusters), axis=0)
    assign_ref[...] = assign
    onehot = (iota_c == assign[None, :]).astype(jnp.float32)
    cnt = jnp.sum(onehot, axis=1, keepdims=True)
    counts_out_ref[...] += jnp.broadcast_to(cnt, counts_out_ref.shape)
    xg_ref[...] = (
        x.reshape(bn // 8, 8, 16, 16).transpose(2, 0, 1, 3)
        .reshape(16, bn // 8, 128))


def _tc_argmin(x, centers):
    n, dim = x.shape
    num_clusters = centers.shape[0]
    bn = 512
    num_blocks = n // bn
    return pl.pallas_call(
        functools.partial(_argmin_body, num_blocks=num_blocks,
                          num_clusters=num_clusters, bn=bn),
        grid=(num_blocks,),
        in_specs=[
            pl.BlockSpec((bn, dim), lambda i: (i, 0)),
            pl.BlockSpec((num_clusters, dim), lambda i: (0, 0)),
        ],
        out_specs=[
            pl.BlockSpec((bn,), lambda i: (i,)),
            pl.BlockSpec((num_clusters, 8), lambda i: (0, 0)),
            pl.BlockSpec((16, bn // 8, 128), lambda i: (0, i, 0)),
        ],
        out_shape=[
            jax.ShapeDtypeStruct((n,), jnp.int32),
            jax.ShapeDtypeStruct((num_clusters, 8), jnp.float32),
            jax.ShapeDtypeStruct((16, n // 8, 128), jnp.float32),
        ],
        scratch_shapes=[pltpu.VMEM((num_clusters, 8), jnp.float32)],
        compiler_params=pltpu.CompilerParams(
            dimension_semantics=("arbitrary",)),
    )(x, centers)


_CHUNK = 1024


def _make_sc_scatter(n, dim, num_clusters):
    info = plsc.get_sparse_core_info()
    nc, ns = info.num_cores, info.num_subcores
    half = n // nc
    rows_half = half // 8
    rows_chunk = _CHUNK // 8
    nchunks = half // _CHUNK
    mesh = plsc.VectorSubcoreMesh(core_axis_name="c", subcore_axis_name="s")

    @functools.partial(
        pl.kernel, mesh=mesh,
        out_type=jax.ShapeDtypeStruct((nc, ns, num_clusters // 8, 128),
                                      jnp.float32),
        scratch_types=[
            pltpu.VMEM((num_clusters // 8, 128), jnp.float32),
            pltpu.VMEM((half,), jnp.int32),
            pltpu.VMEM((2, rows_chunk, 128), jnp.float32),
            pltpu.SemaphoreType.DMA,
            pltpu.SemaphoreType.DMA,
        ],
    )
    def sc_scatter(xg_hbm, a_hbm, sums_out, acc, idx_v, x_v, sem0, sem1):
        c = lax.axis_index("c")
        cb = lax.axis_index("s")
        sems = (sem0, sem1)

        @plsc.parallel_loop(0, num_clusters // 8)
        def _zero(r):
            for jj in range(8):
                acc[r, pl.ds(jj * 16, 16)] = jnp.zeros((16,), jnp.float32)

        pltpu.sync_copy(a_hbm.at[pl.ds(c * half, half)], idx_v)

        def start(k, b):
            return pltpu.async_copy(
                xg_hbm.at[cb, pl.ds(c * rows_half + k * rows_chunk,
                                    rows_chunk)],
                x_v.at[b], sems[b])

        def process(k, b):
            @plsc.parallel_loop(0, _CHUNK // 16, unroll=2)
            def _grp(g):
                avec = idx_v[pl.ds(k * _CHUNK + g * 16, 16)]
                for l in range(16):
                    a = avec[l]
                    plsc.addupdate(
                        acc.at[a >> 3, pl.ds((a & 7) * 16, 16)],
                        x_v[b, g * 2 + l // 8, pl.ds((l % 8) * 16, 16)])

        start(0, 0).wait()

        @pl.loop(0, nchunks // 2)
        def _pair(kk):
            k0 = kk * 2

            @pl.when(k0 + 1 < nchunks)
            def _s1():
                start(k0 + 1, 1)

            process(k0, 0)

            @pl.when(k0 + 1 < nchunks)
            def _w1():
                pltpu.make_async_copy(
                    xg_hbm.at[cb, pl.ds(0, rows_chunk)], x_v.at[1],
                    sem1).wait()

            @pl.when(k0 + 2 < nchunks)
            def _s2():
                start(k0 + 2, 0)

            @pl.when(k0 + 1 < nchunks)
            def _p1():
                process(k0 + 1, 1)

            @pl.when(k0 + 2 < nchunks)
            def _w2():
                pltpu.make_async_copy(
                    xg_hbm.at[cb, pl.ds(0, rows_chunk)], x_v.at[0],
                    sem0).wait()

        pltpu.sync_copy(acc, sums_out.at[c, cb])

    return sc_scatter


def _combine_body(sums_ref, cnt_ref, c_ref, centers_out_ref):
    sums = sums_ref[0] + sums_ref[1]
    counts = cnt_ref[:, 0:1]
    means = sums / jnp.maximum(counts, 1.0)
    centers_out_ref[...] = jnp.where(counts > 0.0, means, c_ref[...])


def _tc_combine(sums, cnt, centers):
    num_clusters, dim = centers.shape
    return pl.pallas_call(
        _combine_body,
        out_shape=jax.ShapeDtypeStruct((num_clusters, dim), jnp.float32),
    )(sums, cnt, centers)


@jax.jit
def kernel(x, cluster_centers):
    n, dim = x.shape
    num_clusters = cluster_centers.shape[0]

    assignments, counts8, xg = _tc_argmin(x, cluster_centers)
    sc_scatter = _make_sc_scatter(n, dim, num_clusters)
    sums4 = sc_scatter(xg, assignments)
    sums2 = jnp.transpose(
        sums4.reshape(2, 16, num_clusters // 8, 8, 16),
        (0, 2, 3, 1, 4)).reshape(2, num_clusters, dim)
    new_centers = _tc_combine(sums2, counts8, cluster_centers)
    return new_centers, counts8[:, 0]


# speedup vs baseline: 2.7693x; 2.7693x over previous
import functools

import jax
import jax.numpy as jnp
from jax.experimental import pallas as pl
from jax.experimental.pallas import tpu as pltpu


def _kmeans_body(x_ref, c_ref, centers_out_ref, counts_out_ref, c2_scr, *,
                 num_blocks, num_clusters, dim, bn):
    i = pl.program_id(0)

    @pl.when(i == 0)
    def _init():
        cc = c_ref[...]
        c2 = jnp.sum(cc * cc, axis=1, keepdims=True)
        c2_scr[...] = jnp.broadcast_to(c2, (num_clusters, 8))
        centers_out_ref[...] = jnp.zeros_like(centers_out_ref)
        counts_out_ref[...] = jnp.zeros_like(counts_out_ref)

    x = x_ref[...]
    scores = jax.lax.dot_general(
        c_ref[...], x, (((1,), (1,)), ((), ())),
        preferred_element_type=jnp.float32)
    val = scores - 0.5 * c2_scr[:, 0:1]
    mx = jnp.max(val, axis=0, keepdims=True)
    iota_c = jax.lax.broadcasted_iota(jnp.int32, (num_clusters, bn), 0)
    assign = jnp.min(jnp.where(val == mx, iota_c, num_clusters),
                     axis=0)
    onehot = (jax.lax.broadcasted_iota(jnp.int32, (num_clusters, bn), 0)
              == assign[None, :]).astype(jnp.float32)
    centers_out_ref[...] += jax.lax.dot_general(
        onehot.astype(jnp.bfloat16), x.astype(jnp.bfloat16),
        (((1,), (0,)), ((), ())),
        preferred_element_type=jnp.float32)
    cnt = jnp.sum(onehot, axis=1, keepdims=True)
    counts_out_ref[...] += jnp.broadcast_to(cnt, (num_clusters, 8))

    @pl.when(i == num_blocks - 1)
    def _finalize():
        counts = counts_out_ref[:, 0:1]
        sums = centers_out_ref[...]
        means = sums / jnp.maximum(counts, 1.0)
        centers_out_ref[...] = jnp.where(counts > 0.0, means, c_ref[...])


@jax.jit
def kernel(x, cluster_centers):
    n, dim = x.shape
    num_clusters = cluster_centers.shape[0]
    bn = 512
    num_blocks = n // bn

    new_centers, counts8 = pl.pallas_call(
        functools.partial(_kmeans_body, num_blocks=num_blocks,
                          num_clusters=num_clusters, dim=dim, bn=bn),
        grid=(num_blocks,),
        in_specs=[
            pl.BlockSpec((bn, dim), lambda i: (i, 0)),
            pl.BlockSpec((num_clusters, dim), lambda i: (0, 0)),
        ],
        out_specs=[
            pl.BlockSpec((num_clusters, dim), lambda i: (0, 0)),
            pl.BlockSpec((num_clusters, 8), lambda i: (0, 0)),
        ],
        out_shape=[
            jax.ShapeDtypeStruct((num_clusters, dim), jnp.float32),
            jax.ShapeDtypeStruct((num_clusters, 8), jnp.float32),
        ],
        scratch_shapes=[pltpu.VMEM((num_clusters, 8), jnp.float32)],
        compiler_params=pltpu.CompilerParams(
            dimension_semantics=("arbitrary",)),
    )(x, cluster_centers)

    return new_centers, counts8[:, 0]


# speedup vs baseline: 3.2348x; 1.1681x over previous
import functools

import jax
import jax.numpy as jnp
from jax.experimental import pallas as pl
from jax.experimental.pallas import tpu as pltpu


def _kmeans_body(x_ref, c_ref, centers_out_ref, counts_out_ref, c2_scr, *,
                 num_blocks, num_clusters, dim, bn):
    i = pl.program_id(0)

    @pl.when(i == 0)
    def _init():
        cc = c_ref[...]
        c2 = jnp.sum(cc * cc, axis=1, keepdims=True)
        c2_scr[...] = jnp.broadcast_to(c2, (num_clusters, 8))
        centers_out_ref[...] = jnp.zeros_like(centers_out_ref)
        counts_out_ref[...] = jnp.zeros_like(counts_out_ref)

    x = x_ref[...]
    scores = jax.lax.dot_general(
        c_ref[...], x, (((1,), (1,)), ((), ())),
        preferred_element_type=jnp.float32)
    val = scores - 0.5 * c2_scr[:, 0:1]
    mx = jnp.max(val, axis=0, keepdims=True)
    onehot = (val == mx).astype(jnp.float32)
    centers_out_ref[...] += jax.lax.dot_general(
        onehot, x, (((1,), (0,)), ((), ())),
        preferred_element_type=jnp.float32)
    cnt = jnp.sum(onehot, axis=1, keepdims=True)
    counts_out_ref[...] += jnp.broadcast_to(cnt, (num_clusters, 8))

    @pl.when(i == num_blocks - 1)
    def _finalize():
        counts = counts_out_ref[:, 0:1]
        sums = centers_out_ref[...]
        means = sums / jnp.maximum(counts, 1.0)
        centers_out_ref[...] = jnp.where(counts > 0.0, means, c_ref[...])


@jax.jit
def kernel(x, cluster_centers):
    n, dim = x.shape
    num_clusters = cluster_centers.shape[0]
    bn = 512
    num_blocks = n // bn

    new_centers, counts8 = pl.pallas_call(
        functools.partial(_kmeans_body, num_blocks=num_blocks,
                          num_clusters=num_clusters, dim=dim, bn=bn),
        grid=(num_blocks,),
        in_specs=[
            pl.BlockSpec((bn, dim), lambda i: (i, 0)),
            pl.BlockSpec((num_clusters, dim), lambda i: (0, 0)),
        ],
        out_specs=[
            pl.BlockSpec((num_clusters, dim), lambda i: (0, 0)),
            pl.BlockSpec((num_clusters, 8), lambda i: (0, 0)),
        ],
        out_shape=[
            jax.ShapeDtypeStruct((num_clusters, dim), jnp.float32),
            jax.ShapeDtypeStruct((num_clusters, 8), jnp.float32),
        ],
        scratch_shapes=[pltpu.VMEM((num_clusters, 8), jnp.float32)],
        compiler_params=pltpu.CompilerParams(
            dimension_semantics=("arbitrary",)),
    )(x, cluster_centers)

    return new_centers, counts8[:, 0]


# speedup vs baseline: 4.3411x; 1.3420x over previous
import functools

import jax
import jax.numpy as jnp
from jax.experimental import pallas as pl
from jax.experimental.pallas import tpu as pltpu


def _kmeans_body(x_ref, c_ref, centers_out_ref, counts_out_ref, c2_scr, *,
                 num_blocks, num_clusters, dim, bn):
    i = pl.program_id(0)

    @pl.when(i == 0)
    def _init():
        cc = c_ref[...]
        c2 = jnp.sum(cc * cc, axis=1, keepdims=True)
        c2_scr[...] = jnp.broadcast_to(c2, (num_clusters, 8))
        centers_out_ref[...] = jnp.zeros_like(centers_out_ref)
        counts_out_ref[...] = jnp.zeros_like(counts_out_ref)

    x = x_ref[...]
    scores = jax.lax.dot_general(
        c_ref[...], x, (((1,), (1,)), ((), ())),
        preferred_element_type=jnp.float32)
    val = scores - 0.5 * c2_scr[:, 0:1]
    mx = jnp.max(val, axis=0, keepdims=True)
    onehot = (val == mx).astype(jnp.float32)
    centers_out_ref[...] += jax.lax.dot_general(
        onehot, x, (((1,), (0,)), ((), ())),
        preferred_element_type=jnp.float32)
    cnt = jnp.sum(onehot, axis=1, keepdims=True)
    counts_out_ref[...] += jnp.broadcast_to(cnt, (num_clusters, 8))

    @pl.when(i == num_blocks - 1)
    def _finalize():
        counts = counts_out_ref[:, 0:1]
        sums = centers_out_ref[...]
        means = sums / jnp.maximum(counts, 1.0)
        centers_out_ref[...] = jnp.where(counts > 0.0, means, c_ref[...])


@jax.jit
def kernel(x, cluster_centers):
    n, dim = x.shape
    num_clusters = cluster_centers.shape[0]
    bn = 1024
    num_blocks = n // bn

    new_centers, counts8 = pl.pallas_call(
        functools.partial(_kmeans_body, num_blocks=num_blocks,
                          num_clusters=num_clusters, dim=dim, bn=bn),
        grid=(num_blocks,),
        in_specs=[
            pl.BlockSpec((bn, dim), lambda i: (i, 0)),
            pl.BlockSpec((num_clusters, dim), lambda i: (0, 0)),
        ],
        out_specs=[
            pl.BlockSpec((num_clusters, dim), lambda i: (0, 0)),
            pl.BlockSpec((num_clusters, 8), lambda i: (0, 0)),
        ],
        out_shape=[
            jax.ShapeDtypeStruct((num_clusters, dim), jnp.float32),
            jax.ShapeDtypeStruct((num_clusters, 8), jnp.float32),
        ],
        scratch_shapes=[pltpu.VMEM((num_clusters, 8), jnp.float32)],
        compiler_params=pltpu.CompilerParams(
            dimension_semantics=("arbitrary",)),
    )(x, cluster_centers)

    return new_centers, counts8[:, 0]


# speedup vs baseline: 5.0576x; 1.1651x over previous
import functools

import jax
import jax.numpy as jnp
from jax.experimental import pallas as pl
from jax.experimental.pallas import tpu as pltpu


def _kmeans_body(x_ref, c_ref, centers_out_ref, counts_out_ref, c2_scr, *,
                 num_blocks, num_clusters, dim, bn):
    i = pl.program_id(0)

    @pl.when(i == 0)
    def _init():
        cc = c_ref[...]
        c2 = jnp.sum(cc * cc, axis=1, keepdims=True)
        c2_scr[...] = jnp.broadcast_to(c2, (num_clusters, 8))
        centers_out_ref[...] = jnp.zeros_like(centers_out_ref)
        counts_out_ref[...] = jnp.zeros_like(counts_out_ref)

    x = x_ref[...]
    scores = jax.lax.dot_general(
        c_ref[...], x, (((1,), (1,)), ((), ())),
        preferred_element_type=jnp.float32)
    val = scores - 0.5 * c2_scr[:, 0:1]
    mx = jnp.max(val, axis=0, keepdims=True)
    onehot = (val == mx).astype(jnp.float32)
    centers_out_ref[...] += jax.lax.dot_general(
        onehot, x, (((1,), (0,)), ((), ())),
        preferred_element_type=jnp.float32)
    cnt = jnp.sum(onehot, axis=1, keepdims=True)
    counts_out_ref[...] += jnp.broadcast_to(cnt, (num_clusters, 8))

    @pl.when(i == num_blocks - 1)
    def _finalize():
        counts = counts_out_ref[:, 0:1]
        sums = centers_out_ref[...]
        means = sums / jnp.maximum(counts, 1.0)
        centers_out_ref[...] = jnp.where(counts > 0.0, means, c_ref[...])


@jax.jit
def kernel(x, cluster_centers):
    n, dim = x.shape
    num_clusters = cluster_centers.shape[0]
    bn = 2048
    num_blocks = n // bn

    new_centers, counts8 = pl.pallas_call(
        functools.partial(_kmeans_body, num_blocks=num_blocks,
                          num_clusters=num_clusters, dim=dim, bn=bn),
        grid=(num_blocks,),
        in_specs=[
            pl.BlockSpec((bn, dim), lambda i: (i, 0)),
            pl.BlockSpec((num_clusters, dim), lambda i: (0, 0)),
        ],
        out_specs=[
            pl.BlockSpec((num_clusters, dim), lambda i: (0, 0)),
            pl.BlockSpec((num_clusters, 8), lambda i: (0, 0)),
        ],
        out_shape=[
            jax.ShapeDtypeStruct((num_clusters, dim), jnp.float32),
            jax.ShapeDtypeStruct((num_clusters, 8), jnp.float32),
        ],
        scratch_shapes=[pltpu.VMEM((num_clusters, 8), jnp.float32)],
        compiler_params=pltpu.CompilerParams(
            dimension_semantics=("arbitrary",)),
    )(x, cluster_centers)

    return new_centers, counts8[:, 0]


# speedup vs baseline: 5.4863x; 1.0848x over previous
import functools

import jax
import jax.numpy as jnp
from jax.experimental import pallas as pl
from jax.experimental.pallas import tpu as pltpu


def _kmeans_body(x_ref, c_ref, centers_out_ref, counts_out_ref, c2_scr, *,
                 num_blocks, num_clusters, dim, bn):
    i = pl.program_id(0)

    @pl.when(i == 0)
    def _init():
        cc = c_ref[...]
        c2 = jnp.sum(cc * cc, axis=1, keepdims=True)
        c2_scr[...] = jnp.broadcast_to(c2, (num_clusters, 8))
        centers_out_ref[...] = jnp.zeros_like(centers_out_ref)
        counts_out_ref[...] = jnp.zeros_like(counts_out_ref)

    x = x_ref[...]
    scores = jax.lax.dot_general(
        c_ref[...], x, (((1,), (1,)), ((), ())),
        preferred_element_type=jnp.float32)
    val = scores - 0.5 * c2_scr[:, 0:1]
    mx = jnp.max(val, axis=0, keepdims=True)
    onehot = (val == mx).astype(jnp.float32)
    centers_out_ref[...] += jax.lax.dot_general(
        onehot, x, (((1,), (0,)), ((), ())),
        preferred_element_type=jnp.float32)
    cnt = jnp.sum(onehot, axis=1, keepdims=True)
    counts_out_ref[...] += jnp.broadcast_to(cnt, (num_clusters, 8))

    @pl.when(i == num_blocks - 1)
    def _finalize():
        counts = counts_out_ref[:, 0:1]
        sums = centers_out_ref[...]
        means = sums / jnp.maximum(counts, 1.0)
        centers_out_ref[...] = jnp.where(counts > 0.0, means, c_ref[...])


@jax.jit
def kernel(x, cluster_centers):
    n, dim = x.shape
    num_clusters = cluster_centers.shape[0]
    bn = 4096
    num_blocks = n // bn

    new_centers, counts8 = pl.pallas_call(
        functools.partial(_kmeans_body, num_blocks=num_blocks,
                          num_clusters=num_clusters, dim=dim, bn=bn),
        grid=(num_blocks,),
        in_specs=[
            pl.BlockSpec((bn, dim), lambda i: (i, 0)),
            pl.BlockSpec((num_clusters, dim), lambda i: (0, 0)),
        ],
        out_specs=[
            pl.BlockSpec((num_clusters, dim), lambda i: (0, 0)),
            pl.BlockSpec((num_clusters, 8), lambda i: (0, 0)),
        ],
        out_shape=[
            jax.ShapeDtypeStruct((num_clusters, dim), jnp.float32),
            jax.ShapeDtypeStruct((num_clusters, 8), jnp.float32),
        ],
        scratch_shapes=[pltpu.VMEM((num_clusters, 8), jnp.float32)],
        compiler_params=pltpu.CompilerParams(
            dimension_semantics=("arbitrary",)),
    )(x, cluster_centers)

    return new_centers, counts8[:, 0]


# speedup vs baseline: 5.6857x; 1.0363x over previous
import functools

import jax
import jax.numpy as jnp
from jax.experimental import pallas as pl
from jax.experimental.pallas import tpu as pltpu


def _kmeans_body(x_ref, c_ref, centers_out_ref, counts_out_ref, c2_scr, *,
                 num_blocks, num_clusters, dim, bn):
    i = pl.program_id(0)

    @pl.when(i == 0)
    def _init():
        cc = c_ref[...]
        c2 = jnp.sum(cc * cc, axis=1, keepdims=True)
        c2_scr[...] = jnp.broadcast_to(c2, (num_clusters, 8))
        centers_out_ref[...] = jnp.zeros_like(centers_out_ref)
        counts_out_ref[...] = jnp.zeros_like(counts_out_ref)

    x = x_ref[...]
    scores = jax.lax.dot_general(
        c_ref[...], x, (((1,), (1,)), ((), ())),
        preferred_element_type=jnp.float32)
    val = scores - 0.5 * c2_scr[:, 0:1]
    mx = jnp.max(val, axis=0, keepdims=True)
    onehot = (val == mx).astype(jnp.float32)
    centers_out_ref[...] += jax.lax.dot_general(
        onehot, x, (((1,), (0,)), ((), ())),
        preferred_element_type=jnp.float32)
    cnt = jnp.sum(onehot, axis=1, keepdims=True)
    counts_out_ref[...] += jnp.broadcast_to(cnt, (num_clusters, 8))

    @pl.when(i == num_blocks - 1)
    def _finalize():
        counts = counts_out_ref[:, 0:1]
        sums = centers_out_ref[...]
        means = sums / jnp.maximum(counts, 1.0)
        centers_out_ref[...] = jnp.where(counts > 0.0, means, c_ref[...])


@jax.jit
def kernel(x, cluster_centers):
    n, dim = x.shape
    num_clusters = cluster_centers.shape[0]
    bn = 8192
    num_blocks = n // bn

    new_centers, counts8 = pl.pallas_call(
        functools.partial(_kmeans_body, num_blocks=num_blocks,
                          num_clusters=num_clusters, dim=dim, bn=bn),
        grid=(num_blocks,),
        in_specs=[
            pl.BlockSpec((bn, dim), lambda i: (i, 0)),
            pl.BlockSpec((num_clusters, dim), lambda i: (0, 0)),
        ],
        out_specs=[
            pl.BlockSpec((num_clusters, dim), lambda i: (0, 0)),
            pl.BlockSpec((num_clusters, 8), lambda i: (0, 0)),
        ],
        out_shape=[
            jax.ShapeDtypeStruct((num_clusters, dim), jnp.float32),
            jax.ShapeDtypeStruct((num_clusters, 8), jnp.float32),
        ],
        scratch_shapes=[pltpu.VMEM((num_clusters, 8), jnp.float32)],
        compiler_params=pltpu.CompilerParams(
            dimension_semantics=("arbitrary",)),
    )(x, cluster_centers)

    return new_centers, counts8[:, 0]
